# baseline (speedup 1.0000x reference)
import functools

import jax
import jax.numpy as jnp
from jax import lax
from jax.experimental import pallas as pl
from jax.experimental.pallas import tpu as pltpu
from jax.experimental.pallas import tpu_sc as plsc

B = 16384
GMF_DIM = 16
MLP_DIM = 64
N_ROWS = 1000000

NC = 2
NS = 16
NW = NC * NS
BPW = B // NW
L = 16
CH = 128
NCH = BPW // CH

TBLK = 1024
TGRID = (N_ROWS + TBLK - 1) // TBLK
M_ROWS = TGRID * 512
G_ROWS = TGRID * 128

_CONTRACT0 = (((0,), (0,)), ((), ()))


def _pack_body(muT_ref, miT_ref, guT_ref, giT_ref, e64_ref, e16_ref,
               mu2_ref, mi2_ref, gu2_ref, gi2_ref):
    e64 = e64_ref[...]
    e16 = e16_ref[...]

    def t(x, e):
        return lax.dot_general(x, e, _CONTRACT0,
                               preferred_element_type=jnp.float32)

    for src, dst in ((muT_ref, mu2_ref), (miT_ref, mi2_ref)):
        x = src[...]
        dst[...] = jnp.concatenate(
            [t(x[:, :512], e64), t(x[:, 512:], e64)], axis=1)
    for src, dst in ((guT_ref, gu2_ref), (giT_ref, gi2_ref)):
        x = src[...]
        dst[...] = jnp.concatenate(
            [t(x[:, 128 * h:128 * (h + 1)], e16) for h in range(8)], axis=1)


def _tc_pack(muT, miT, guT, giT):
    e64 = jnp.eye(MLP_DIM, dtype=jnp.float32)
    e16 = jnp.eye(GMF_DIM, dtype=jnp.float32)
    return pl.pallas_call(
        _pack_body,
        grid=(TGRID,),
        in_specs=[
            pl.BlockSpec((MLP_DIM, TBLK), lambda i: (0, i)),
            pl.BlockSpec((MLP_DIM, TBLK), lambda i: (0, i)),
            pl.BlockSpec((GMF_DIM, TBLK), lambda i: (0, i)),
            pl.BlockSpec((GMF_DIM, TBLK), lambda i: (0, i)),
            pl.BlockSpec((MLP_DIM, MLP_DIM), lambda i: (0, 0)),
            pl.BlockSpec((GMF_DIM, GMF_DIM), lambda i: (0, 0)),
        ],
        out_specs=[
            pl.BlockSpec((512, 128), lambda i: (i, 0)),
            pl.BlockSpec((512, 128), lambda i: (i, 0)),
            pl.BlockSpec((128, 128), lambda i: (i, 0)),
            pl.BlockSpec((128, 128), lambda i: (i, 0)),
        ],
        out_shape=[
            jax.ShapeDtypeStruct((M_ROWS, 128), jnp.float32),
            jax.ShapeDtypeStruct((M_ROWS, 128), jnp.float32),
            jax.ShapeDtypeStruct((G_ROWS, 128), jnp.float32),
            jax.ShapeDtypeStruct((G_ROWS, 128), jnp.float32),
        ],
    )(muT, miT, guT, giT, e64, e16)


def _sc_gather(uid, iid, gu2, gi2, mu2, mi2):
    mesh = plsc.VectorSubcoreMesh(core_axis_name="c", subcore_axis_name="s")

    @functools.partial(
        pl.kernel,
        mesh=mesh,
        out_type=[
            jax.ShapeDtypeStruct((B, 128), jnp.float32),
            jax.ShapeDtypeStruct((B, 128), jnp.float32),
        ],
        scratch_types=[
            pltpu.VMEM((BPW,), jnp.int32),
            pltpu.VMEM((BPW,), jnp.int32),
            pltpu.VMEM((CH, 128), jnp.float32),
            pltpu.VMEM((CH, 128), jnp.float32),
            pltpu.VMEM((CH, 128), jnp.float32),
            pltpu.VMEM((CH, 128), jnp.float32),
            pltpu.VMEM((CH, 128), jnp.float32),
            pltpu.SemaphoreType.DMA,
        ],
    )
    def body(uid_hbm, iid_hbm, gu_hbm, gi_hbm, mu_hbm, mi_hbm,
             out_mlp, out_gmf,
             uidx_v, iidx_v, gub, gib, mub, mib, stage, sem):
        wid = lax.axis_index("s") * NC + lax.axis_index("c")
        base = wid * BPW
        pltpu.sync_copy(uid_hbm.at[pl.ds(base, BPW)], uidx_v)
        pltpu.sync_copy(iid_hbm.at[pl.ds(base, BPW)], iidx_v)

        for ch in range(NCH):
            def gstep(g, carry, ch=ch):
                uvec = uidx_v[pl.ds(ch * CH + g * L, L)]
                ivec = iidx_v[pl.ds(ch * CH + g * L, L)]
                for l in range(L):
                    u = uvec[l]
                    i = ivec[l]
                    k = g * L + l
                    dst = pl.ds(k, 1)
                    rm_u = ((u >> 10) << 9) + (u & 511)
                    rm_i = ((i >> 10) << 9) + (i & 511)
                    rg_u = ((u >> 10) << 7) + (u & 127)
                    rg_i = ((i >> 10) << 7) + (i & 127)
                    pltpu.async_copy(mu_hbm.at[pl.ds(rm_u, 1)], mub.at[dst], sem)
                    pltpu.async_copy(mi_hbm.at[pl.ds(rm_i, 1)], mib.at[dst], sem)
                    pltpu.async_copy(gu_hbm.at[pl.ds(rg_u, 1)], gub.at[dst], sem)
                    pltpu.async_copy(gi_hbm.at[pl.ds(rg_i, 1)], gib.at[dst], sem)
                return carry

            lax.fori_loop(0, CH // L, gstep, 0)
            pltpu.make_async_copy(mu_hbm.at[pl.ds(0, CH)], mub, sem).wait()
            pltpu.make_async_copy(mu_hbm.at[pl.ds(0, CH)], mib, sem).wait()
            pltpu.make_async_copy(gu_hbm.at[pl.ds(0, CH)], gub, sem).wait()
            pltpu.make_async_copy(gu_hbm.at[pl.ds(0, CH)], gib, sem).wait()

            def mstep(g, carry, ch=ch):
                uvec = uidx_v[pl.ds(ch * CH + g * L, L)]
                ivec = iidx_v[pl.ds(ch * CH + g * L, L)]
                for l in range(L):
                    k = g * L + l
                    uoff = ((uvec[l] >> 9) & 1) * 64
                    ioff = ((ivec[l] >> 9) & 1) * 64
                    for c in range(MLP_DIM // L):
                        stage[k, pl.ds(c * L, L)] = mub[k, pl.ds(uoff + c * L, L)]
                        stage[k, pl.ds(64 + c * L, L)] = mib[k, pl.ds(ioff + c * L, L)]
                return carry

            lax.fori_loop(0, CH // L, mstep, 0)
            pltpu.sync_copy(stage, out_mlp.at[pl.ds(base + ch * CH, CH)])

            def pstep(g, carry, ch=ch):
                uvec = uidx_v[pl.ds(ch * CH + g * L, L)]
                ivec = iidx_v[pl.ds(ch * CH + g * L, L)]
                for l in range(L):
                    k = g * L + l
                    uoff = ((uvec[l] >> 7) & 7) * 16
                    ioff = ((ivec[l] >> 7) & 7) * 16
                    a = gub[k, pl.ds(uoff, L)]
                    b = gib[k, pl.ds(ioff, L)]
                    stage[k, pl.ds(0, L)] = a * b
                return carry

            lax.fori_loop(0, CH // L, pstep, 0)
            pltpu.sync_copy(stage, out_gmf.at[pl.ds(base + ch * CH, CH)])

    return body(uid, iid, gu2, gi2, mu2, mi2)


def _tc_body(mlp_ref, gmf_ref, W0_ref, b0_ref, W1_ref, b1_ref,
             W2_ref, b2_ref, Wout_ref, bout_ref, out_ref):
    x = jnp.dot(mlp_ref[...], W0_ref[...], preferred_element_type=jnp.float32)
    x = jnp.maximum(x + b0_ref[...], 0.0)
    x = jnp.maximum(jnp.dot(x, W1_ref[...], preferred_element_type=jnp.float32) + b1_ref[...], 0.0)
    x = jnp.maximum(jnp.dot(x, W2_ref[...], preferred_element_type=jnp.float32) + b2_ref[...], 0.0)
    z = (jnp.dot(gmf_ref[:, :GMF_DIM], Wout_ref[:GMF_DIM, :], preferred_element_type=jnp.float32)
         + jnp.dot(x, Wout_ref[GMF_DIM:, :], preferred_element_type=jnp.float32))
    out_ref[...] = jax.nn.sigmoid(z + bout_ref[...])


def _tc_compute(mlp2d, gmf2d, W0, b0, W1, b1, W2, b2, Wout, bout):
    BLK = 2048
    grid = (B // BLK,)
    full = lambda shape: pl.BlockSpec(shape, lambda i: (0, 0))
    return pl.pallas_call(
        _tc_body,
        grid=grid,
        in_specs=[
            pl.BlockSpec((BLK, 128), lambda i: (i, 0)),
            pl.BlockSpec((BLK, 128), lambda i: (i, 0)),
            full((2 * MLP_DIM, 64)),
            full((1, 64)),
            full((64, 32)),
            full((1, 32)),
            full((32, GMF_DIM)),
            full((1, GMF_DIM)),
            full((32, 1)),
            full((1, 1)),
        ],
        out_specs=pl.BlockSpec((BLK, 1), lambda i: (i, 0)),
        out_shape=jax.ShapeDtypeStruct((B, 1), jnp.float32),
    )(mlp2d, gmf2d, W0, b0, W1, b1, W2, b2, Wout, bout)


def kernel(user_id, item_id, gmf_user_table, gmf_item_table, mlp_user_table,
           mlp_item_table, W0, b0, W1, b1, W2, b2, Wout, bout):
    uid = user_id.astype(jnp.int32)
    iid = item_id.astype(jnp.int32)
    mu2, mi2, gu2, gi2 = _tc_pack(mlp_user_table.T, mlp_item_table.T,
                                  gmf_user_table.T, gmf_item_table.T)
    out_mlp, out_gmf = _sc_gather(uid, iid, gu2, gi2, mu2, mi2)
    return _tc_compute(out_mlp, out_gmf, W0, b0.reshape(1, -1), W1,
                       b1.reshape(1, -1), W2, b2.reshape(1, -1), Wout,
                       bout.reshape(1, -1))


# speedup vs baseline: 1.2526x; 1.2526x over previous
import functools

import jax
import jax.numpy as jnp
from jax import lax
from jax.experimental import pallas as pl
from jax.experimental.pallas import tpu as pltpu
from jax.experimental.pallas import tpu_sc as plsc

B = 16384
GMF_DIM = 16
MLP_DIM = 64
N_ROWS = 1000000

NC = 2
NS = 16
NW = NC * NS
BPW = B // NW
L = 16
CH = 128
NCH = BPW // CH

TBLK = 4096
TGRID = (N_ROWS + TBLK - 1) // TBLK
M_ROWS = TGRID * (TBLK // 2)
G_ROWS = TGRID * (TBLK // 8)

_CONTRACT0 = (((0,), (0,)), ((), ()))


def _pack_body(muT_ref, miT_ref, guT_ref, giT_ref, e64_ref, e16_ref,
               mu2_ref, mi2_ref, gu2_ref, gi2_ref):
    e64 = e64_ref[...]
    e16 = e16_ref[...]

    def t(x, e):
        return lax.dot_general(x, e, _CONTRACT0,
                               preferred_element_type=jnp.float32)

    for src, dst in ((muT_ref, mu2_ref), (miT_ref, mi2_ref)):
        x = src[...]
        dst[...] = jnp.concatenate(
            [t(x[:, :TBLK // 2], e64), t(x[:, TBLK // 2:], e64)], axis=1)
    for src, dst in ((guT_ref, gu2_ref), (giT_ref, gi2_ref)):
        x = src[...]
        dst[...] = jnp.concatenate(
            [t(x[:, (TBLK // 8) * h:(TBLK // 8) * (h + 1)], e16) for h in range(8)], axis=1)


def _tc_pack(muT, miT, guT, giT):
    e64 = jnp.eye(MLP_DIM, dtype=jnp.float32)
    e16 = jnp.eye(GMF_DIM, dtype=jnp.float32)
    return pl.pallas_call(
        _pack_body,
        grid=(TGRID,),
        in_specs=[
            pl.BlockSpec((MLP_DIM, TBLK), lambda i: (0, i)),
            pl.BlockSpec((MLP_DIM, TBLK), lambda i: (0, i)),
            pl.BlockSpec((GMF_DIM, TBLK), lambda i: (0, i)),
            pl.BlockSpec((GMF_DIM, TBLK), lambda i: (0, i)),
            pl.BlockSpec((MLP_DIM, MLP_DIM), lambda i: (0, 0)),
            pl.BlockSpec((GMF_DIM, GMF_DIM), lambda i: (0, 0)),
        ],
        out_specs=[
            pl.BlockSpec((TBLK // 2, 128), lambda i: (i, 0)),
            pl.BlockSpec((TBLK // 2, 128), lambda i: (i, 0)),
            pl.BlockSpec((TBLK // 8, 128), lambda i: (i, 0)),
            pl.BlockSpec((TBLK // 8, 128), lambda i: (i, 0)),
        ],
        out_shape=[
            jax.ShapeDtypeStruct((M_ROWS, 128), jnp.float32),
            jax.ShapeDtypeStruct((M_ROWS, 128), jnp.float32),
            jax.ShapeDtypeStruct((G_ROWS, 128), jnp.float32),
            jax.ShapeDtypeStruct((G_ROWS, 128), jnp.float32),
        ],
    )(muT, miT, guT, giT, e64, e16)


def _sc_gather(uid, iid, gu2, gi2, mu2, mi2):
    mesh = plsc.VectorSubcoreMesh(core_axis_name="c", subcore_axis_name="s")

    @functools.partial(
        pl.kernel,
        mesh=mesh,
        out_type=[
            jax.ShapeDtypeStruct((B, 128), jnp.float32),
            jax.ShapeDtypeStruct((B, 128), jnp.float32),
        ],
        scratch_types=[
            pltpu.VMEM((BPW,), jnp.int32),
            pltpu.VMEM((BPW,), jnp.int32),
            pltpu.VMEM((CH, 128), jnp.float32),
            pltpu.VMEM((CH, 128), jnp.float32),
            pltpu.VMEM((CH, 128), jnp.float32),
            pltpu.VMEM((CH, 128), jnp.float32),
            pltpu.VMEM((CH, 128), jnp.float32),
            pltpu.SemaphoreType.DMA,
        ],
    )
    def body(uid_hbm, iid_hbm, gu_hbm, gi_hbm, mu_hbm, mi_hbm,
             out_mlp, out_gmf,
             uidx_v, iidx_v, gub, gib, mub, mib, stage, sem):
        wid = lax.axis_index("s") * NC + lax.axis_index("c")
        base = wid * BPW
        pltpu.sync_copy(uid_hbm.at[pl.ds(base, BPW)], uidx_v)
        pltpu.sync_copy(iid_hbm.at[pl.ds(base, BPW)], iidx_v)

        for ch in range(NCH):
            def gstep(g, carry, ch=ch):
                uvec = uidx_v[pl.ds(ch * CH + g * L, L)]
                ivec = iidx_v[pl.ds(ch * CH + g * L, L)]
                for l in range(L):
                    u = uvec[l]
                    i = ivec[l]
                    k = g * L + l
                    dst = pl.ds(k, 1)
                    rm_u = ((u >> 12) << 11) + (u & 2047)
                    rm_i = ((i >> 12) << 11) + (i & 2047)
                    rg_u = ((u >> 12) << 9) + (u & 511)
                    rg_i = ((i >> 12) << 9) + (i & 511)
                    pltpu.async_copy(mu_hbm.at[pl.ds(rm_u, 1)], mub.at[dst], sem)
                    pltpu.async_copy(mi_hbm.at[pl.ds(rm_i, 1)], mib.at[dst], sem)
                    pltpu.async_copy(gu_hbm.at[pl.ds(rg_u, 1)], gub.at[dst], sem)
                    pltpu.async_copy(gi_hbm.at[pl.ds(rg_i, 1)], gib.at[dst], sem)
                return carry

            lax.fori_loop(0, CH // L, gstep, 0)
            pltpu.make_async_copy(mu_hbm.at[pl.ds(0, CH)], mub, sem).wait()
            pltpu.make_async_copy(mu_hbm.at[pl.ds(0, CH)], mib, sem).wait()
            pltpu.make_async_copy(gu_hbm.at[pl.ds(0, CH)], gub, sem).wait()
            pltpu.make_async_copy(gu_hbm.at[pl.ds(0, CH)], gib, sem).wait()

            def mstep(g, carry, ch=ch):
                uvec = uidx_v[pl.ds(ch * CH + g * L, L)]
                ivec = iidx_v[pl.ds(ch * CH + g * L, L)]
                for l in range(L):
                    k = g * L + l
                    uoff = ((uvec[l] >> 11) & 1) * 64
                    ioff = ((ivec[l] >> 11) & 1) * 64
                    for c in range(MLP_DIM // L):
                        stage[k, pl.ds(c * L, L)] = mub[k, pl.ds(uoff + c * L, L)]
                        stage[k, pl.ds(64 + c * L, L)] = mib[k, pl.ds(ioff + c * L, L)]
                return carry

            lax.fori_loop(0, CH // L, mstep, 0)
            pltpu.sync_copy(stage, out_mlp.at[pl.ds(base + ch * CH, CH)])

            def pstep(g, carry, ch=ch):
                uvec = uidx_v[pl.ds(ch * CH + g * L, L)]
                ivec = iidx_v[pl.ds(ch * CH + g * L, L)]
                for l in range(L):
                    k = g * L + l
                    uoff = ((uvec[l] >> 9) & 7) * 16
                    ioff = ((ivec[l] >> 9) & 7) * 16
                    a = gub[k, pl.ds(uoff, L)]
                    b = gib[k, pl.ds(ioff, L)]
                    stage[k, pl.ds(0, L)] = a * b
                return carry

            lax.fori_loop(0, CH // L, pstep, 0)
            pltpu.sync_copy(stage, out_gmf.at[pl.ds(base + ch * CH, CH)])

    return body(uid, iid, gu2, gi2, mu2, mi2)


def _tc_body(mlp_ref, gmf_ref, W0_ref, b0_ref, W1_ref, b1_ref,
             W2_ref, b2_ref, Wout_ref, bout_ref, out_ref):
    x = jnp.dot(mlp_ref[...], W0_ref[...], preferred_element_type=jnp.float32)
    x = jnp.maximum(x + b0_ref[...], 0.0)
    x = jnp.maximum(jnp.dot(x, W1_ref[...], preferred_element_type=jnp.float32) + b1_ref[...], 0.0)
    x = jnp.maximum(jnp.dot(x, W2_ref[...], preferred_element_type=jnp.float32) + b2_ref[...], 0.0)
    z = (jnp.dot(gmf_ref[:, :GMF_DIM], Wout_ref[:GMF_DIM, :], preferred_element_type=jnp.float32)
         + jnp.dot(x, Wout_ref[GMF_DIM:, :], preferred_element_type=jnp.float32))
    out_ref[...] = jax.nn.sigmoid(z + bout_ref[...])


def _tc_compute(mlp2d, gmf2d, W0, b0, W1, b1, W2, b2, Wout, bout):
    BLK = 2048
    grid = (B // BLK,)
    full = lambda shape: pl.BlockSpec(shape, lambda i: (0, 0))
    return pl.pallas_call(
        _tc_body,
        grid=grid,
        in_specs=[
            pl.BlockSpec((BLK, 128), lambda i: (i, 0)),
            pl.BlockSpec((BLK, 128), lambda i: (i, 0)),
            full((2 * MLP_DIM, 64)),
            full((1, 64)),
            full((64, 32)),
            full((1, 32)),
            full((32, GMF_DIM)),
            full((1, GMF_DIM)),
            full((32, 1)),
            full((1, 1)),
        ],
        out_specs=pl.BlockSpec((BLK, 1), lambda i: (i, 0)),
        out_shape=jax.ShapeDtypeStruct((B, 1), jnp.float32),
    )(mlp2d, gmf2d, W0, b0, W1, b1, W2, b2, Wout, bout)


def kernel(user_id, item_id, gmf_user_table, gmf_item_table, mlp_user_table,
           mlp_item_table, W0, b0, W1, b1, W2, b2, Wout, bout):
    uid = user_id.astype(jnp.int32)
    iid = item_id.astype(jnp.int32)
    mu2, mi2, gu2, gi2 = _tc_pack(mlp_user_table.T, mlp_item_table.T,
                                  gmf_user_table.T, gmf_item_table.T)
    out_mlp, out_gmf = _sc_gather(uid, iid, gu2, gi2, mu2, mi2)
    return _tc_compute(out_mlp, out_gmf, W0, b0.reshape(1, -1), W1,
                       b1.reshape(1, -1), W2, b2.reshape(1, -1), Wout,
                       bout.reshape(1, -1))


# speedup vs baseline: 1.2561x; 1.0028x over previous
import functools

import jax
import jax.numpy as jnp
from jax import lax
from jax.experimental import pallas as pl
from jax.experimental.pallas import tpu as pltpu
from jax.experimental.pallas import tpu_sc as plsc

B = 16384
GMF_DIM = 16
MLP_DIM = 64
N_ROWS = 1000000

NC = 2
NS = 16
NW = NC * NS
BPW = B // NW
L = 16
CH = 128
NCH = BPW // CH

TBLK = 8192
TGRID = (N_ROWS + TBLK - 1) // TBLK
M_ROWS = TGRID * (TBLK // 2)
G_ROWS = TGRID * (TBLK // 8)

_CONTRACT0 = (((0,), (0,)), ((), ()))


def _pack_body(muT_ref, miT_ref, guT_ref, giT_ref, e64_ref, e16_ref,
               mu2_ref, mi2_ref, gu2_ref, gi2_ref):
    e64 = e64_ref[...]
    e16 = e16_ref[...]

    def t(x, e):
        return lax.dot_general(x, e, _CONTRACT0,
                               preferred_element_type=jnp.float32)

    for src, dst in ((muT_ref, mu2_ref), (miT_ref, mi2_ref)):
        x = src[...]
        dst[...] = jnp.concatenate(
            [t(x[:, :TBLK // 2], e64), t(x[:, TBLK // 2:], e64)], axis=1)
    for src, dst in ((guT_ref, gu2_ref), (giT_ref, gi2_ref)):
        x = src[...]
        dst[...] = jnp.concatenate(
            [t(x[:, (TBLK // 8) * h:(TBLK // 8) * (h + 1)], e16) for h in range(8)], axis=1)


def _tc_pack(muT, miT, guT, giT):
    e64 = jnp.eye(MLP_DIM, dtype=jnp.float32)
    e16 = jnp.eye(GMF_DIM, dtype=jnp.float32)
    return pl.pallas_call(
        _pack_body,
        grid=(TGRID,),
        in_specs=[
            pl.BlockSpec((MLP_DIM, TBLK), lambda i: (0, i)),
            pl.BlockSpec((MLP_DIM, TBLK), lambda i: (0, i)),
            pl.BlockSpec((GMF_DIM, TBLK), lambda i: (0, i)),
            pl.BlockSpec((GMF_DIM, TBLK), lambda i: (0, i)),
            pl.BlockSpec((MLP_DIM, MLP_DIM), lambda i: (0, 0)),
            pl.BlockSpec((GMF_DIM, GMF_DIM), lambda i: (0, 0)),
        ],
        out_specs=[
            pl.BlockSpec((TBLK // 2, 128), lambda i: (i, 0)),
            pl.BlockSpec((TBLK // 2, 128), lambda i: (i, 0)),
            pl.BlockSpec((TBLK // 8, 128), lambda i: (i, 0)),
            pl.BlockSpec((TBLK // 8, 128), lambda i: (i, 0)),
        ],
        out_shape=[
            jax.ShapeDtypeStruct((M_ROWS, 128), jnp.float32),
            jax.ShapeDtypeStruct((M_ROWS, 128), jnp.float32),
            jax.ShapeDtypeStruct((G_ROWS, 128), jnp.float32),
            jax.ShapeDtypeStruct((G_ROWS, 128), jnp.float32),
        ],
    )(muT, miT, guT, giT, e64, e16)


def _sc_gather(uid, iid, gu2, gi2, mu2, mi2):
    mesh = plsc.VectorSubcoreMesh(core_axis_name="c", subcore_axis_name="s")

    @functools.partial(
        pl.kernel,
        mesh=mesh,
        out_type=[
            jax.ShapeDtypeStruct((B, 128), jnp.float32),
            jax.ShapeDtypeStruct((B, 128), jnp.float32),
        ],
        scratch_types=[
            pltpu.VMEM((BPW,), jnp.int32),
            pltpu.VMEM((BPW,), jnp.int32),
            pltpu.VMEM((CH, 128), jnp.float32),
            pltpu.VMEM((CH, 128), jnp.float32),
            pltpu.VMEM((CH, 128), jnp.float32),
            pltpu.VMEM((CH, 128), jnp.float32),
            pltpu.VMEM((CH, 128), jnp.float32),
            pltpu.SemaphoreType.DMA,
        ],
    )
    def body(uid_hbm, iid_hbm, gu_hbm, gi_hbm, mu_hbm, mi_hbm,
             out_mlp, out_gmf,
             uidx_v, iidx_v, gub, gib, mub, mib, stage, sem):
        wid = lax.axis_index("s") * NC + lax.axis_index("c")
        base = wid * BPW
        pltpu.sync_copy(uid_hbm.at[pl.ds(base, BPW)], uidx_v)
        pltpu.sync_copy(iid_hbm.at[pl.ds(base, BPW)], iidx_v)

        for ch in range(NCH):
            def gstep(g, carry, ch=ch):
                uvec = uidx_v[pl.ds(ch * CH + g * L, L)]
                ivec = iidx_v[pl.ds(ch * CH + g * L, L)]
                for l in range(L):
                    u = uvec[l]
                    i = ivec[l]
                    k = g * L + l
                    dst = pl.ds(k, 1)
                    rm_u = ((u >> 13) << 12) + (u & 4095)
                    rm_i = ((i >> 13) << 12) + (i & 4095)
                    rg_u = ((u >> 13) << 10) + (u & 1023)
                    rg_i = ((i >> 13) << 10) + (i & 1023)
                    pltpu.async_copy(mu_hbm.at[pl.ds(rm_u, 1)], mub.at[dst], sem)
                    pltpu.async_copy(mi_hbm.at[pl.ds(rm_i, 1)], mib.at[dst], sem)
                    pltpu.async_copy(gu_hbm.at[pl.ds(rg_u, 1)], gub.at[dst], sem)
                    pltpu.async_copy(gi_hbm.at[pl.ds(rg_i, 1)], gib.at[dst], sem)
                return carry

            lax.fori_loop(0, CH // L, gstep, 0)
            pltpu.make_async_copy(mu_hbm.at[pl.ds(0, CH)], mub, sem).wait()
            pltpu.make_async_copy(mu_hbm.at[pl.ds(0, CH)], mib, sem).wait()
            pltpu.make_async_copy(gu_hbm.at[pl.ds(0, CH)], gub, sem).wait()
            pltpu.make_async_copy(gu_hbm.at[pl.ds(0, CH)], gib, sem).wait()

            def mstep(g, carry, ch=ch):
                uvec = uidx_v[pl.ds(ch * CH + g * L, L)]
                ivec = iidx_v[pl.ds(ch * CH + g * L, L)]
                for l in range(L):
                    k = g * L + l
                    uoff = ((uvec[l] >> 12) & 1) * 64
                    ioff = ((ivec[l] >> 12) & 1) * 64
                    for c in range(MLP_DIM // L):
                        stage[k, pl.ds(c * L, L)] = mub[k, pl.ds(uoff + c * L, L)]
                        stage[k, pl.ds(64 + c * L, L)] = mib[k, pl.ds(ioff + c * L, L)]
                return carry

            lax.fori_loop(0, CH // L, mstep, 0)
            pltpu.sync_copy(stage, out_mlp.at[pl.ds(base + ch * CH, CH)])

            def pstep(g, carry, ch=ch):
                uvec = uidx_v[pl.ds(ch * CH + g * L, L)]
                ivec = iidx_v[pl.ds(ch * CH + g * L, L)]
                for l in range(L):
                    k = g * L + l
                    uoff = ((uvec[l] >> 10) & 7) * 16
                    ioff = ((ivec[l] >> 10) & 7) * 16
                    a = gub[k, pl.ds(uoff, L)]
                    b = gib[k, pl.ds(ioff, L)]
                    stage[k, pl.ds(0, L)] = a * b
                return carry

            lax.fori_loop(0, CH // L, pstep, 0)
            pltpu.sync_copy(stage, out_gmf.at[pl.ds(base + ch * CH, CH)])

    return body(uid, iid, gu2, gi2, mu2, mi2)


def _tc_body(mlp_ref, gmf_ref, W0_ref, b0_ref, W1_ref, b1_ref,
             W2_ref, b2_ref, Wout_ref, bout_ref, out_ref):
    x = jnp.dot(mlp_ref[...], W0_ref[...], preferred_element_type=jnp.float32)
    x = jnp.maximum(x + b0_ref[...], 0.0)
    x = jnp.maximum(jnp.dot(x, W1_ref[...], preferred_element_type=jnp.float32) + b1_ref[...], 0.0)
    x = jnp.maximum(jnp.dot(x, W2_ref[...], preferred_element_type=jnp.float32) + b2_ref[...], 0.0)
    z = (jnp.dot(gmf_ref[:, :GMF_DIM], Wout_ref[:GMF_DIM, :], preferred_element_type=jnp.float32)
         + jnp.dot(x, Wout_ref[GMF_DIM:, :], preferred_element_type=jnp.float32))
    out_ref[...] = jax.nn.sigmoid(z + bout_ref[...])


def _tc_compute(mlp2d, gmf2d, W0, b0, W1, b1, W2, b2, Wout, bout):
    BLK = 2048
    grid = (B // BLK,)
    full = lambda shape: pl.BlockSpec(shape, lambda i: (0, 0))
    return pl.pallas_call(
        _tc_body,
        grid=grid,
        in_specs=[
            pl.BlockSpec((BLK, 128), lambda i: (i, 0)),
            pl.BlockSpec((BLK, 128), lambda i: (i, 0)),
            full((2 * MLP_DIM, 64)),
            full((1, 64)),
            full((64, 32)),
            full((1, 32)),
            full((32, GMF_DIM)),
            full((1, GMF_DIM)),
            full((32, 1)),
            full((1, 1)),
        ],
        out_specs=pl.BlockSpec((BLK, 1), lambda i: (i, 0)),
        out_shape=jax.ShapeDtypeStruct((B, 1), jnp.float32),
    )(mlp2d, gmf2d, W0, b0, W1, b1, W2, b2, Wout, bout)


def kernel(user_id, item_id, gmf_user_table, gmf_item_table, mlp_user_table,
           mlp_item_table, W0, b0, W1, b1, W2, b2, Wout, bout):
    uid = user_id.astype(jnp.int32)
    iid = item_id.astype(jnp.int32)
    mu2, mi2, gu2, gi2 = _tc_pack(mlp_user_table.T, mlp_item_table.T,
                                  gmf_user_table.T, gmf_item_table.T)
    out_mlp, out_gmf = _sc_gather(uid, iid, gu2, gi2, mu2, mi2)
    return _tc_compute(out_mlp, out_gmf, W0, b0.reshape(1, -1), W1,
                       b1.reshape(1, -1), W2, b2.reshape(1, -1), Wout,
                       bout.reshape(1, -1))
